# baseline (speedup 1.0000x reference)
import functools

import numpy as np
import jax
import jax.numpy as jnp
from jax import lax
from jax.experimental import pallas as pl
from jax.experimental.pallas import tpu as pltpu
from jax.experimental.pallas import tpu_sc as plsc

B, T, S = 4096, 10000, 128

NUM_CORES = 2
NUM_SUBCORES = 16
NUM_WORKERS = NUM_CORES * NUM_SUBCORES
ROWS_PER_W = B // NUM_WORKERS


def _subsample_taps():
    max_logspace = np.log10(T - 1)
    idx = np.round(np.logspace(0.0, max_logspace, S, endpoint=True), 1).astype(int)
    idx[0] = 0
    return idx.astype(np.int32)


_TAPS = _subsample_taps()
_FLAT_IDX = (np.arange(B, dtype=np.int64)[:, None] * T + _TAPS[None, :]).astype(
    np.int32
)


def _sc_gather_body(xf_hbm, fidx_hbm, out_hbm, idx_v, data_v, sem):
    wid = lax.axis_index("s") * NUM_CORES + lax.axis_index("c")
    base = wid * ROWS_PER_W
    pltpu.sync_copy(fidx_hbm.at[pl.ds(base, ROWS_PER_W)], idx_v)

    def fire(r, carry):
        pltpu.async_copy(xf_hbm.at[idx_v.at[r]], data_v.at[r], sem)
        return carry

    lax.fori_loop(0, ROWS_PER_W, fire, 0)
    pltpu.make_async_copy(out_hbm.at[pl.ds(base, ROWS_PER_W)], data_v, sem).wait()
    pltpu.sync_copy(data_v, out_hbm.at[pl.ds(base, ROWS_PER_W)])


_sc_gather = functools.partial(
    pl.kernel,
    mesh=plsc.VectorSubcoreMesh(core_axis_name="c", subcore_axis_name="s"),
    out_type=jax.ShapeDtypeStruct((B, S), jnp.float32),
    scratch_types=[
        pltpu.VMEM((ROWS_PER_W, S), jnp.int32),
        pltpu.VMEM((ROWS_PER_W, S), jnp.float32),
        pltpu.SemaphoreType.DMA,
    ],
)(_sc_gather_body)


@jax.jit
def kernel(x):
    xf = x.reshape(-1)
    fidx = jnp.asarray(_FLAT_IDX)
    return _sc_gather(xf, fidx)
